# Initial kernel scaffold; baseline (speedup 1.0000x reference)
import jax
import jax.numpy as jnp
from jax.experimental import pallas as pl


def kernel(x, edge_index, edge_weight, W_fc0, b_fc0, conv_weights, W_fstr, b_fstr):
    raise NotImplementedError("write your pallas kernel here")


# speedup vs baseline: 2.2957x; 2.2957x over previous
import functools
import math

import jax
import jax.numpy as jnp
from jax import lax
from jax.experimental import pallas as pl
from jax.experimental.pallas import tpu as pltpu
from jax.experimental.pallas import tpu_sc as plsc

N = 10000
E = 160000
D = 256
L = 8
LAMDA = 0.5
ALPHA = 0.1

DH = D // 2
NS = 16
NPAD = 10112
K = 80
ET = E // NS
NCH = ET // K
RPT = NPAD // NS

BN = RPT
_PREC = lax.Precision.HIGHEST


def _spmm_body(h2_ref, src_ref, dst_ref, w_ref, zeros_ref, out_ref,
               src_v, dst_v, w_v, rows_v, acc, sem):
    c = lax.axis_index("c")
    s = lax.axis_index("s")
    rbase = s * RPT
    pltpu.sync_copy(zeros_ref.at[pl.ds(rbase, RPT)], acc.at[pl.ds(rbase, RPT)])
    plsc.subcore_barrier()

    cN = c * NPAD

    def chunk(i, carry):
        base = s * ET + i * K
        pltpu.sync_copy(src_ref.at[pl.ds(base, K)], src_v)
        pltpu.sync_copy(dst_ref.at[pl.ds(base, K)], dst_v)
        pltpu.sync_copy(w_ref.at[pl.ds(base, K)], w_v)
        for m in range(K // 16):
            sl = pl.ds(m * 16, 16)
            src_v[sl] = src_v[sl] + cN
        pltpu.async_copy(h2_ref.at[src_v], rows_v, sem).wait()

        def edge(k, carry2):
            wkv = plsc.load_gather(w_v, [jnp.full((16,), k, jnp.int32)])
            for j in range(DH // 16):
                sl = pl.ds(j * 16, 16)
                rows_v[k, sl] = rows_v[k, sl] * wkv
            return carry2

        lax.fori_loop(0, K, edge, 0)
        pltpu.sync_copy(rows_v, acc.at[dst_v], add=True)
        return carry

    lax.fori_loop(0, NCH, chunk, 0)
    plsc.subcore_barrier()
    pltpu.sync_copy(acc.at[pl.ds(rbase, RPT)],
                    out_ref.at[pl.ds(cN + rbase, RPT)])


_spmm = functools.partial(
    pl.kernel,
    out_type=jax.ShapeDtypeStruct((2 * NPAD, DH), jnp.float32),
    mesh=plsc.VectorSubcoreMesh(core_axis_name="c", subcore_axis_name="s"),
    compiler_params=pltpu.CompilerParams(needs_layout_passes=False),
    scratch_types=[
        pltpu.VMEM((K,), jnp.int32),
        pltpu.VMEM((K,), jnp.int32),
        pltpu.VMEM((K,), jnp.float32),
        pltpu.VMEM((K, DH), jnp.float32),
        pltpu.VMEM_SHARED((NPAD, DH), jnp.float32),
        pltpu.SemaphoreType.DMA,
    ],
)(_spmm_body)


def _pre_body(x_ref, w_ref, b_ref, out_ref):
    acc = lax.dot_general(x_ref[...], w_ref[...], (((1,), (1,)), ((), ())),
                          preferred_element_type=jnp.float32, precision=_PREC)
    h = jnp.maximum(acc + b_ref[...], 0.0)
    out_ref[0] = h[:, :DH]
    out_ref[1] = h[:, DH:]


def _layer_body(theta, hi_ref, h_ref, h0_ref, w_ref, out_ref):
    sup_a = (1.0 - ALPHA) * hi_ref[0] + ALPHA * h0_ref[0]
    sup_b = (1.0 - ALPHA) * hi_ref[1] + ALPHA * h0_ref[1]
    sup = jnp.concatenate([sup_a, sup_b], axis=1)
    mm = lax.dot_general(sup, w_ref[...], (((1,), (0,)), ((), ())),
                         preferred_element_type=jnp.float32, precision=_PREC)
    h_full = jnp.concatenate([h_ref[0], h_ref[1]], axis=1)
    out = theta * mm + (1.0 - theta) * sup + h_full
    out = jnp.maximum(out, 0.0)
    out_ref[0] = out[:, :DH]
    out_ref[1] = out[:, DH:]


def _post_body(h_ref, w_ref, b_ref, out_ref):
    h_full = jnp.concatenate([h_ref[0], h_ref[1]], axis=1)
    acc = lax.dot_general(h_full, w_ref[...], (((1,), (1,)), ((), ())),
                          preferred_element_type=jnp.float32, precision=_PREC)
    out_ref[...] = acc + b_ref[...]


_GRID = (NPAD // BN,)
_spec_full_w = pl.BlockSpec((D, D), lambda i: (0, 0))
_spec_bias = pl.BlockSpec((1, D), lambda i: (0, 0))
_spec_rows = pl.BlockSpec((BN, D), lambda i: (i, 0))
_spec_planes = pl.BlockSpec((2, BN, DH), lambda i: (0, i, 0))

_pre = pl.pallas_call(
    _pre_body,
    grid=_GRID,
    in_specs=[_spec_rows, _spec_full_w, _spec_bias],
    out_specs=_spec_planes,
    out_shape=jax.ShapeDtypeStruct((2, NPAD, DH), jnp.float32),
)

_post = pl.pallas_call(
    _post_body,
    grid=_GRID,
    in_specs=[_spec_planes, _spec_full_w, _spec_bias],
    out_specs=_spec_rows,
    out_shape=jax.ShapeDtypeStruct((NPAD, D), jnp.float32),
)


def _make_layer(theta):
    return pl.pallas_call(
        functools.partial(_layer_body, theta),
        grid=_GRID,
        in_specs=[_spec_planes, _spec_planes, _spec_planes, _spec_full_w],
        out_specs=_spec_planes,
        out_shape=jax.ShapeDtypeStruct((2, NPAD, DH), jnp.float32),
    )


def kernel(x, edge_index, edge_weight, W_fc0, b_fc0, conv_weights,
           W_fstr, b_fstr):
    dst = edge_index[0]
    src = edge_index[1]
    zeros = jnp.zeros((NPAD, DH), jnp.float32)
    xp = jnp.pad(x, ((0, NPAD - N), (0, 0)))

    h = _pre(xp, W_fc0, b_fc0.reshape(1, D))
    h0 = h
    for i in range(L):
        theta = min(1.0, math.log(LAMDA / (i + 1) + 1.0))
        hi2 = _spmm(h.reshape(2 * NPAD, DH), src, dst, edge_weight, zeros)
        h = _make_layer(theta)(hi2.reshape(2, NPAD, DH), h, h0,
                               conv_weights[i])
    return _post(h, W_fstr, b_fstr.reshape(1, D))[:N]


# speedup vs baseline: 5.7186x; 2.4910x over previous
import functools
import math

import jax
import jax.numpy as jnp
from jax import lax
from jax.experimental import pallas as pl
from jax.experimental.pallas import tpu as pltpu
from jax.experimental.pallas import tpu_sc as plsc

N = 10000
E = 160000
D = 256
L = 8
LAMDA = 0.5
ALPHA = 0.1

DH = D // 2
NS = 16
NPAD = 10112
K = 80
ET = E // NS
NCH = ET // K
RPT = NPAD // NS

NEB = 6
NRB = 3

BN = RPT
_PREC = lax.Precision.HIGHEST


def _spmm_body(h2_ref, ep_ref, zeros_ref, out_ref, *scr):
    ebufs = scr[0:NEB]
    rows = scr[NEB:NEB + NRB]
    acc = scr[NEB + NRB]
    esems = scr[NEB + NRB + 1:NEB + NRB + 1 + NEB]
    gsems = scr[NEB + NRB + 1 + NEB:NEB + NRB + 1 + NEB + NRB]
    ssems = scr[NEB + NRB + 1 + NEB + NRB:]

    c = lax.axis_index("c")
    s = lax.axis_index("s")
    rbase = s * RPT
    coff = c * NPAD

    def load_ebuf(ch, slot):
        pltpu.async_copy(ep_ref.at[s * NCH + ch], ebufs[slot], esems[slot])

    def prep_gather(ch, slot6, slot3):
        pltpu.make_async_copy(ep_ref.at[s * NCH + ch], ebufs[slot6],
                              esems[slot6]).wait()
        eb = ebufs[slot6]
        for m in range(K // 16):
            sl = pl.ds(m * 16, 16)
            eb[0, sl] = eb[0, sl] + coff
        pltpu.async_copy(h2_ref.at[eb.at[0]], rows[slot3], gsems[slot3])

    def mul_chunk(slot6, slot3):
        rb = rows[slot3]
        eb = ebufs[slot6]

        def edge(k, carry):
            wi = plsc.load_gather(eb, [jnp.full((16,), 2, jnp.int32),
                                       jnp.full((16,), k, jnp.int32)])
            wkv = plsc.bitcast(wi, jnp.float32)
            for j in range(DH // 16):
                sl = pl.ds(j * 16, 16)
                rb[k, sl] = rb[k, sl] * wkv
            return carry

        lax.fori_loop(0, K, edge, 0, unroll=4)

    pltpu.sync_copy(zeros_ref.at[pl.ds(rbase, RPT)], acc.at[pl.ds(rbase, RPT)])
    for ch in range(4):
        load_ebuf(ch, ch)
    for ch in range(2):
        prep_gather(ch, ch, ch)
    plsc.subcore_barrier()

    def six(t, carry):
        for u in range(6):
            ch = 6 * t + u

            @pl.when(ch >= 1)
            def _():
                pltpu.make_async_copy(
                    rows[(u - 1) % NRB],
                    acc.at[ebufs[(u - 1) % NEB].at[1]],
                    ssems[(u - 1) % NRB]).wait()

            @pl.when(ch + 4 < NCH)
            def _():
                load_ebuf(ch + 4, (u + 4) % NEB)

            @pl.when(ch + 2 < NCH)
            def _():
                prep_gather(ch + 2, (u + 2) % NEB, (u + 2) % NRB)

            @pl.when(ch < NCH)
            def _():
                pltpu.make_async_copy(h2_ref.at[ebufs[u % NEB].at[0]],
                                      rows[u % NRB], gsems[u % NRB]).wait()
                mul_chunk(u % NEB, u % NRB)
                pltpu.async_copy(rows[u % NRB],
                                 acc.at[ebufs[u % NEB].at[1]],
                                 ssems[u % NRB], add=True)
        return carry

    lax.fori_loop(0, (NCH + 6) // 6, six, 0)

    plsc.subcore_barrier()
    pltpu.sync_copy(acc.at[pl.ds(rbase, RPT)],
                    out_ref.at[pl.ds(coff + rbase, RPT)])


_spmm = functools.partial(
    pl.kernel,
    out_type=jax.ShapeDtypeStruct((2 * NPAD, DH), jnp.float32),
    mesh=plsc.VectorSubcoreMesh(core_axis_name="c", subcore_axis_name="s"),
    compiler_params=pltpu.CompilerParams(needs_layout_passes=False),
    scratch_types=(
        [pltpu.VMEM((3, K), jnp.int32)] * NEB
        + [pltpu.VMEM((K, DH), jnp.float32)] * NRB
        + [pltpu.VMEM_SHARED((NPAD, DH), jnp.float32)]
        + [pltpu.SemaphoreType.DMA] * (NEB + NRB + NRB)
    ),
)(_spmm_body)


def _pre_body(x_ref, w_ref, b_ref, out_ref):
    acc = lax.dot_general(x_ref[...], w_ref[...], (((1,), (1,)), ((), ())),
                          preferred_element_type=jnp.float32, precision=_PREC)
    h = jnp.maximum(acc + b_ref[...], 0.0)
    out_ref[0] = h[:, :DH]
    out_ref[1] = h[:, DH:]


def _layer_body(theta, hi_ref, h_ref, h0_ref, w_ref, out_ref):
    sup_a = (1.0 - ALPHA) * hi_ref[0] + ALPHA * h0_ref[0]
    sup_b = (1.0 - ALPHA) * hi_ref[1] + ALPHA * h0_ref[1]
    sup = jnp.concatenate([sup_a, sup_b], axis=1)
    mm = lax.dot_general(sup, w_ref[...], (((1,), (0,)), ((), ())),
                         preferred_element_type=jnp.float32, precision=_PREC)
    h_full = jnp.concatenate([h_ref[0], h_ref[1]], axis=1)
    out = theta * mm + (1.0 - theta) * sup + h_full
    out = jnp.maximum(out, 0.0)
    out_ref[0] = out[:, :DH]
    out_ref[1] = out[:, DH:]


def _post_body(h_ref, w_ref, b_ref, out_ref):
    h_full = jnp.concatenate([h_ref[0], h_ref[1]], axis=1)
    acc = lax.dot_general(h_full, w_ref[...], (((1,), (1,)), ((), ())),
                          preferred_element_type=jnp.float32, precision=_PREC)
    out_ref[...] = acc + b_ref[...]


_GRID = (NPAD // BN,)
_spec_full_w = pl.BlockSpec((D, D), lambda i: (0, 0))
_spec_bias = pl.BlockSpec((1, D), lambda i: (0, 0))
_spec_rows = pl.BlockSpec((BN, D), lambda i: (i, 0))
_spec_planes = pl.BlockSpec((2, BN, DH), lambda i: (0, i, 0))

_pre = pl.pallas_call(
    _pre_body,
    grid=_GRID,
    in_specs=[_spec_rows, _spec_full_w, _spec_bias],
    out_specs=_spec_planes,
    out_shape=jax.ShapeDtypeStruct((2, NPAD, DH), jnp.float32),
)

_post = pl.pallas_call(
    _post_body,
    grid=_GRID,
    in_specs=[_spec_planes, _spec_full_w, _spec_bias],
    out_specs=_spec_rows,
    out_shape=jax.ShapeDtypeStruct((NPAD, D), jnp.float32),
)


def _make_layer(theta):
    return pl.pallas_call(
        functools.partial(_layer_body, theta),
        grid=_GRID,
        in_specs=[_spec_planes, _spec_planes, _spec_planes, _spec_full_w],
        out_specs=_spec_planes,
        out_shape=jax.ShapeDtypeStruct((2, NPAD, DH), jnp.float32),
    )


def kernel(x, edge_index, edge_weight, W_fc0, b_fc0, conv_weights,
           W_fstr, b_fstr):
    dst = edge_index[0]
    src = edge_index[1]
    wbits = lax.bitcast_convert_type(edge_weight, jnp.int32)
    epack = jnp.stack([src.reshape(NS * NCH, K), dst.reshape(NS * NCH, K),
                       wbits.reshape(NS * NCH, K)], axis=1)
    zeros = jnp.zeros((NPAD, DH), jnp.float32)
    xp = jnp.pad(x, ((0, NPAD - N), (0, 0)))

    h = _pre(xp, W_fc0, b_fc0.reshape(1, D))
    h0 = h
    for i in range(L):
        theta = min(1.0, math.log(LAMDA / (i + 1) + 1.0))
        hi2 = _spmm(h.reshape(2 * NPAD, DH), epack, zeros)
        h = _make_layer(theta)(hi2.reshape(2, NPAD, DH), h, h0,
                               conv_weights[i])
    return _post(h, W_fstr, b_fstr.reshape(1, D))[:N]


# speedup vs baseline: 7.2456x; 1.2670x over previous
import functools
import math

import jax
import jax.numpy as jnp
from jax import lax
from jax.experimental import pallas as pl
from jax.experimental.pallas import tpu as pltpu
from jax.experimental.pallas import tpu_sc as plsc

N = 10000
E = 160000
D = 256
L = 8
LAMDA = 0.5
ALPHA = 0.1

DH = D // 2
NS = 16
NPAD = 10112
K = 80
ET = E // NS
NCH = ET // K
RPT = NPAD // NS

NEB = 6
NRB = 3

BN = RPT
_PREC = lax.Precision.HIGHEST
_GD = lax.GatherDimensionNumbers(offset_dims=(), collapsed_slice_dims=(0,),
                                 start_index_map=(0,))


def _spmm_body(h2_ref, ep_ref, zeros_ref, out_ref, *scr):
    ebufs = scr[0:NEB]
    rows = scr[NEB:NEB + NRB]
    acc = scr[NEB + NRB]
    esems = scr[NEB + NRB + 1:NEB + NRB + 1 + NEB]
    gsems = scr[NEB + NRB + 1 + NEB:NEB + NRB + 1 + NEB + NRB]
    ssems = scr[NEB + NRB + 1 + NEB + NRB:]

    c = lax.axis_index("c")
    s = lax.axis_index("s")
    rbase = s * RPT
    coff = c * NPAD

    def load_ebuf(ch, slot):
        pltpu.async_copy(ep_ref.at[s * NCH + ch], ebufs[slot], esems[slot])

    def prep_gather(ch, slot6, slot3):
        pltpu.make_async_copy(ep_ref.at[s * NCH + ch], ebufs[slot6],
                              esems[slot6]).wait()
        eb = ebufs[slot6]
        for m in range(K // 16):
            sl = pl.ds(m * 16, 16)
            eb[0, sl] = eb[0, sl] + coff
        pltpu.async_copy(h2_ref.at[eb.at[0]], rows[slot3], gsems[slot3])

    def mul_chunk(slot6, slot3):
        rb = rows[slot3]
        eb = ebufs[slot6]

        def grp(m, carry):
            mb = pl.multiple_of(m * 16, 16)
            w16 = plsc.bitcast(eb[2, pl.ds(mb, 16)], jnp.float32)
            for e in range(16):
                wkv = lax.gather(
                    w16, jnp.full((16, 1), e, jnp.int32), _GD, (1,),
                    mode=lax.GatherScatterMode.PROMISE_IN_BOUNDS)
                k = mb + e
                for j in range(DH // 16):
                    sl = pl.ds(j * 16, 16)
                    rb[k, sl] = rb[k, sl] * wkv
            return carry

        lax.fori_loop(0, K // 16, grp, 0)

    pltpu.sync_copy(zeros_ref.at[pl.ds(rbase, RPT)], acc.at[pl.ds(rbase, RPT)])
    for ch in range(4):
        load_ebuf(ch, ch)
    for ch in range(2):
        prep_gather(ch, ch, ch)
    plsc.subcore_barrier()

    def six(t, carry):
        for u in range(6):
            ch = 6 * t + u

            @pl.when(ch >= 1)
            def _():
                pltpu.make_async_copy(
                    rows[(u - 1) % NRB],
                    acc.at[ebufs[(u - 1) % NEB].at[1]],
                    ssems[(u - 1) % NRB]).wait()

            @pl.when(ch + 4 < NCH)
            def _():
                load_ebuf(ch + 4, (u + 4) % NEB)

            @pl.when(ch + 2 < NCH)
            def _():
                prep_gather(ch + 2, (u + 2) % NEB, (u + 2) % NRB)

            @pl.when(ch < NCH)
            def _():
                pltpu.make_async_copy(h2_ref.at[ebufs[u % NEB].at[0]],
                                      rows[u % NRB], gsems[u % NRB]).wait()
                mul_chunk(u % NEB, u % NRB)
                pltpu.async_copy(rows[u % NRB],
                                 acc.at[ebufs[u % NEB].at[1]],
                                 ssems[u % NRB], add=True)
        return carry

    lax.fori_loop(0, (NCH + 6) // 6, six, 0)

    plsc.subcore_barrier()
    pltpu.sync_copy(acc.at[pl.ds(rbase, RPT)],
                    out_ref.at[pl.ds(coff + rbase, RPT)])


_spmm = functools.partial(
    pl.kernel,
    out_type=jax.ShapeDtypeStruct((2 * NPAD, DH), jnp.float32),
    mesh=plsc.VectorSubcoreMesh(core_axis_name="c", subcore_axis_name="s"),
    compiler_params=pltpu.CompilerParams(needs_layout_passes=False),
    scratch_types=(
        [pltpu.VMEM((3, K), jnp.int32)] * NEB
        + [pltpu.VMEM((K, DH), jnp.float32)] * NRB
        + [pltpu.VMEM_SHARED((NPAD, DH), jnp.float32)]
        + [pltpu.SemaphoreType.DMA] * (NEB + NRB + NRB)
    ),
)(_spmm_body)


def _pre_body(x_ref, w_ref, b_ref, out_ref):
    acc = lax.dot_general(x_ref[...], w_ref[...], (((1,), (1,)), ((), ())),
                          preferred_element_type=jnp.float32, precision=_PREC)
    h = jnp.maximum(acc + b_ref[...], 0.0)
    out_ref[0] = h[:, :DH]
    out_ref[1] = h[:, DH:]


def _layer_body(theta, hi_ref, h_ref, h0_ref, w_ref, out_ref):
    sup_a = (1.0 - ALPHA) * hi_ref[0] + ALPHA * h0_ref[0]
    sup_b = (1.0 - ALPHA) * hi_ref[1] + ALPHA * h0_ref[1]
    sup = jnp.concatenate([sup_a, sup_b], axis=1)
    mm = lax.dot_general(sup, w_ref[...], (((1,), (0,)), ((), ())),
                         preferred_element_type=jnp.float32, precision=_PREC)
    h_full = jnp.concatenate([h_ref[0], h_ref[1]], axis=1)
    out = theta * mm + (1.0 - theta) * sup + h_full
    out = jnp.maximum(out, 0.0)
    out_ref[0] = out[:, :DH]
    out_ref[1] = out[:, DH:]


def _post_body(h_ref, w_ref, b_ref, out_ref):
    h_full = jnp.concatenate([h_ref[0], h_ref[1]], axis=1)
    acc = lax.dot_general(h_full, w_ref[...], (((1,), (1,)), ((), ())),
                          preferred_element_type=jnp.float32, precision=_PREC)
    out_ref[...] = acc + b_ref[...]


_GRID = (NPAD // BN,)
_spec_full_w = pl.BlockSpec((D, D), lambda i: (0, 0))
_spec_bias = pl.BlockSpec((1, D), lambda i: (0, 0))
_spec_rows = pl.BlockSpec((BN, D), lambda i: (i, 0))
_spec_planes = pl.BlockSpec((2, BN, DH), lambda i: (0, i, 0))

_pre = pl.pallas_call(
    _pre_body,
    grid=_GRID,
    in_specs=[_spec_rows, _spec_full_w, _spec_bias],
    out_specs=_spec_planes,
    out_shape=jax.ShapeDtypeStruct((2, NPAD, DH), jnp.float32),
)

_post = pl.pallas_call(
    _post_body,
    grid=_GRID,
    in_specs=[_spec_planes, _spec_full_w, _spec_bias],
    out_specs=_spec_rows,
    out_shape=jax.ShapeDtypeStruct((NPAD, D), jnp.float32),
)


def _make_layer(theta):
    return pl.pallas_call(
        functools.partial(_layer_body, theta),
        grid=_GRID,
        in_specs=[_spec_planes, _spec_planes, _spec_planes, _spec_full_w],
        out_specs=_spec_planes,
        out_shape=jax.ShapeDtypeStruct((2, NPAD, DH), jnp.float32),
    )


def kernel(x, edge_index, edge_weight, W_fc0, b_fc0, conv_weights,
           W_fstr, b_fstr):
    dst = edge_index[0]
    src = edge_index[1]
    wbits = lax.bitcast_convert_type(edge_weight, jnp.int32)
    epack = jnp.stack([src.reshape(NS * NCH, K), dst.reshape(NS * NCH, K),
                       wbits.reshape(NS * NCH, K)], axis=1)
    zeros = jnp.zeros((NPAD, DH), jnp.float32)
    xp = jnp.pad(x, ((0, NPAD - N), (0, 0)))

    h = _pre(xp, W_fc0, b_fc0.reshape(1, D))
    h0 = h
    for i in range(L):
        theta = min(1.0, math.log(LAMDA / (i + 1) + 1.0))
        hi2 = _spmm(h.reshape(2 * NPAD, DH), epack, zeros)
        h = _make_layer(theta)(hi2.reshape(2, NPAD, DH), h, h0,
                               conv_weights[i])
    return _post(h, W_fstr, b_fstr.reshape(1, D))[:N]
